# baseline (speedup 1.0000x reference)
import functools
import math

import jax
import jax.numpy as jnp
from jax import lax
from jax.experimental import pallas as pl
from jax.experimental.pallas import tpu as pltpu
from jax.experimental.pallas import tpu_sc as plsc

_H = 4
_CH = 300
_N = 200
_E = 3200
_SCALE = 1.0 / math.sqrt(_CH)
_NEG = -1e30


_RS = 25
_VCHUNKS = _E // 16
_NW = 208


def _build_stripe(ei_hbm, a_hbm, ei_v, acc_v, lo, nrows):
    pltpu.sync_copy(ei_hbm, ei_v)
    zeros = jnp.zeros((16,), jnp.float32)
    ones = jnp.ones((16,), jnp.float32)
    for r in range(nrows):
        for j in range(_NW // 16):
            acc_v[r, pl.ds(j * 16, 16)] = zeros

    def body(i, carry):
        s16 = ei_v[0, pl.ds(i * 16, 16)]
        d16 = ei_v[1, pl.ds(i * 16, 16)]
        m = (d16 >= lo) & (d16 < lo + nrows)
        row = jnp.where(m, d16 - lo, 0)
        plsc.addupdate_scatter(acc_v, [row, s16], ones, mask=m)
        return carry

    lax.fori_loop(0, _VCHUNKS, body, 0)
    pltpu.sync_copy(acc_v.at[pl.ds(0, nrows), pl.ds(0, _N)],
                    a_hbm.at[pl.ds(lo, nrows)])


def _adjacency_sc(ei1, ei2):
    mesh = plsc.VectorSubcoreMesh(core_axis_name="c", subcore_axis_name="s")

    @functools.partial(
        pl.kernel, mesh=mesh,
        out_type=[jax.ShapeDtypeStruct((_N, _N), jnp.float32),
                  jax.ShapeDtypeStruct((_N, _N), jnp.float32)],
        scratch_types=[pltpu.VMEM((2, _E), jnp.int32),
                       pltpu.VMEM((_RS, _NW), jnp.float32)],
        compiler_params=pltpu.CompilerParams(use_tc_tiling_on_sc=False,
                                             needs_layout_passes=False),
    )
    def adj_kernel(ei1_hbm, ei2_hbm, a1_hbm, a2_hbm, ei_v, acc_v):
        c = lax.axis_index("c")
        s = lax.axis_index("s")
        lo = s * _RS

        @pl.when(s < 8)
        def _():
            @pl.when(c == 0)
            def _():
                _build_stripe(ei1_hbm, a1_hbm, ei_v, acc_v, lo, _RS)

            @pl.when(c == 1)
            def _():
                _build_stripe(ei2_hbm, a2_hbm, ei_v, acc_v, lo, _RS)

    return adj_kernel(ei1, ei2)


def _proj(x, w_ref, b_ref):
    b = b_ref[...].reshape(1, -1)
    return jnp.dot(x, w_ref[...], preferred_element_type=jnp.float32) + b


def _attn_masked(q, k, v, a):
    mask = a > 0.0
    acc = jnp.zeros((_N, _CH), jnp.float32)
    for h in range(_H):
        sl = slice(h * _CH, (h + 1) * _CH)
        qh, kh, vh = q[:, sl], k[:, sl], v[:, sl]
        s = lax.dot_general(qh, kh, (((1,), (1,)), ((), ())),
                            preferred_element_type=jnp.float32) * _SCALE
        sm = jnp.where(mask, s, _NEG)
        amax = jnp.max(sm, axis=1, keepdims=True)
        amax = jnp.where(amax <= _NEG * 0.5, 0.0, amax)
        ex = a * jnp.exp(jnp.where(mask, s - amax, _NEG))
        den = jnp.sum(ex, axis=1, keepdims=True)
        o = jnp.dot(ex, vh, preferred_element_type=jnp.float32) / (den + 1e-16)
        acc = acc + o
    return acc * (1.0 / _H)


def _attn_dense(q, k, v):
    acc = jnp.zeros((_N, _CH), jnp.float32)
    for h in range(_H):
        sl = slice(h * _CH, (h + 1) * _CH)
        qh, kh, vh = q[:, sl], k[:, sl], v[:, sl]
        s = lax.dot_general(qh, kh, (((1,), (1,)), ((), ())),
                            preferred_element_type=jnp.float32) * _SCALE
        amax = jnp.max(s, axis=1, keepdims=True)
        ex = jnp.exp(s - amax)
        den = jnp.sum(ex, axis=1, keepdims=True)
        o = jnp.dot(ex, vh, preferred_element_type=jnp.float32) / (den + 1e-16)
        acc = acc + o
    return acc * (1.0 / _H)


def _conv_self(x, a, wq, bq, wk, bk, wv, bv, ws, bs):
    q = _proj(x, wq, bq)
    k = _proj(x, wk, bk)
    v = _proj(x, wv, bv)
    o = _attn_masked(q, k, v, a)
    return o + _proj(x, ws, bs)


def _conv_cross(xd, xs, wq, bq, wk, bk, wv, bv, ws, bs):
    q = _proj(xd, wq, bq)
    k = _proj(xs, wk, bk)
    v = _proj(xs, wv, bv)
    o = _attn_dense(q, k, v)
    return o + _proj(xd, ws, bs)


def _l2norm(x):
    n = jnp.sqrt(jnp.sum(x * x, axis=1, keepdims=True))
    return x / jnp.maximum(n, 1e-12)


def _body(x1_ref, x2_ref, a1_ref, a2_ref, *refs):
    cp = refs[:32]
    mp = refs[32:40]
    x1p_ref, x2p_ref, out_ref = refs[40:]

    x1 = x1_ref[...]
    x2 = x2_ref[...]
    a1 = a1_ref[...]
    a2 = a2_ref[...]

    x1 = _conv_self(x1, a1, *cp[0:8])
    x2 = _conv_self(x2, a2, *cp[8:16])
    x1c = _conv_cross(x1, x2, *cp[16:24])
    x2c = _conv_cross(x2, x1, *cp[24:32])
    x1n = _l2norm(x1c)
    x2n = _l2norm(x2c)

    x1p = jnp.mean(x1n, axis=0, keepdims=True)
    x2p = jnp.mean(x2n, axis=0, keepdims=True)
    h = jnp.concatenate([x1p, x2p], axis=1)
    h = jnp.maximum(jnp.dot(h, mp[0][...], preferred_element_type=jnp.float32)
                    + mp[1][...].reshape(1, -1), 0.0)
    h = jnp.maximum(jnp.dot(h, mp[2][...], preferred_element_type=jnp.float32)
                    + mp[3][...].reshape(1, -1), 0.0)
    h = jnp.maximum(jnp.dot(h, mp[4][...], preferred_element_type=jnp.float32)
                    + mp[5][...].reshape(1, -1), 0.0)
    z = jnp.dot(h, mp[6][...], preferred_element_type=jnp.float32) + mp[7][...].reshape(1, -1)
    o = 1.0 / (1.0 + jnp.exp(-z))

    x1p_ref[...] = x1p.reshape(_CH)
    x2p_ref[...] = x2p.reshape(_CH)
    out_ref[...] = o.reshape(1)


def _conv_args(p):
    return [p['Wq'], p['bq'], p['Wk'], p['bk'],
            p['Wv'], p['bv'], p['Ws'], p['bs']]


def kernel(x_1, x_2_pos, edge_index_1, edge_index_2_pos, edge_attr_1,
           edge_attr_2_pos, params):
    lp = params['layers'][0]
    m = params['mlp']
    a1, a2 = _adjacency_sc(edge_index_1.astype(jnp.int32),
                           edge_index_2_pos.astype(jnp.int32))
    args = [x_1, x_2_pos, a1, a2]
    for name in ('text_self', 'graph_self', 'text_cross', 'graph_cross'):
        args.extend(_conv_args(lp[name]))
    args.extend([m['W1'], m['b1'], m['W2'], m['b2'],
                 m['W3'], m['b3'], m['W4'], m['b4']])

    x1p, x2p, out = pl.pallas_call(
        _body,
        out_shape=[
            jax.ShapeDtypeStruct((_CH,), jnp.float32),
            jax.ShapeDtypeStruct((_CH,), jnp.float32),
            jax.ShapeDtypeStruct((1,), jnp.float32),
        ],
        compiler_params=pltpu.CompilerParams(
            vmem_limit_bytes=100 * 1024 * 1024),
    )(*args)
    return x1p, x2p, out


# speedup vs baseline: 1.0494x; 1.0494x over previous
import functools
import math

import jax
import jax.numpy as jnp
from jax import lax
from jax.experimental import pallas as pl
from jax.experimental.pallas import tpu as pltpu
from jax.experimental.pallas import tpu_sc as plsc

_H = 4
_CH = 300
_N = 200
_E = 3200
_SCALE = 1.0 / math.sqrt(_CH)
_NEG = -1e30


_RS = 24
_VCHUNKS = _E // 16
_NW = 256


def _build_stripe(ei_hbm, a_hbm, ei_v, acc_v, lo, nrows):
    pltpu.sync_copy(ei_hbm, ei_v)
    zeros = jnp.zeros((16,), jnp.float32)
    ones = jnp.ones((16,), jnp.float32)
    for r in range(nrows):
        for j in range(_NW // 16):
            acc_v[r, pl.ds(j * 16, 16)] = zeros

    def body(i, carry):
        s16 = ei_v[0, pl.ds(i * 16, 16)]
        d16 = ei_v[1, pl.ds(i * 16, 16)]
        m = (d16 >= lo) & (d16 < lo + nrows)
        row = jnp.where(m, d16 - lo, 0)
        plsc.addupdate_scatter(acc_v, [row, s16], ones, mask=m)
        return carry

    lax.fori_loop(0, _VCHUNKS, body, 0)
    pltpu.sync_copy(acc_v.at[pl.ds(0, nrows)], a_hbm.at[pl.ds(lo, nrows)])


def _adjacency_sc(ei1, ei2):
    mesh = plsc.VectorSubcoreMesh(core_axis_name="c", subcore_axis_name="s")

    @functools.partial(
        pl.kernel, mesh=mesh,
        out_type=[jax.ShapeDtypeStruct((_N, _NW), jnp.float32),
                  jax.ShapeDtypeStruct((_N, _NW), jnp.float32)],
        scratch_types=[pltpu.VMEM((2, _E), jnp.int32),
                       pltpu.VMEM((_RS, _NW), jnp.float32)],
        compiler_params=pltpu.CompilerParams(needs_layout_passes=False),
    )
    def adj_kernel(ei1_hbm, ei2_hbm, a1_hbm, a2_hbm, ei_v, acc_v):
        c = lax.axis_index("c")
        s = lax.axis_index("s")
        lo = pl.multiple_of(s * _RS, 8)

        def graph(ei_hbm, a_hbm):
            @pl.when(s < 8)
            def _():
                _build_stripe(ei_hbm, a_hbm, ei_v, acc_v, lo, _RS)

            @pl.when(s == 8)
            def _():
                _build_stripe(ei_hbm, a_hbm, ei_v, acc_v, lo, _N - 8 * _RS)

        @pl.when(c == 0)
        def _():
            graph(ei1_hbm, a1_hbm)

        @pl.when(c == 1)
        def _():
            graph(ei2_hbm, a2_hbm)

    return adj_kernel(ei1, ei2)


def _proj(x, w_ref, b_ref):
    b = b_ref[...].reshape(1, -1)
    return jnp.dot(x, w_ref[...], preferred_element_type=jnp.float32) + b


def _attn_masked(q, k, v, a):
    mask = a > 0.0
    acc = jnp.zeros((_N, _CH), jnp.float32)
    for h in range(_H):
        sl = slice(h * _CH, (h + 1) * _CH)
        qh, kh, vh = q[:, sl], k[:, sl], v[:, sl]
        s = lax.dot_general(qh, kh, (((1,), (1,)), ((), ())),
                            preferred_element_type=jnp.float32) * _SCALE
        sm = jnp.where(mask, s, _NEG)
        amax = jnp.max(sm, axis=1, keepdims=True)
        amax = jnp.where(amax <= _NEG * 0.5, 0.0, amax)
        ex = a * jnp.exp(jnp.where(mask, s - amax, _NEG))
        den = jnp.sum(ex, axis=1, keepdims=True)
        o = jnp.dot(ex, vh, preferred_element_type=jnp.float32) / (den + 1e-16)
        acc = acc + o
    return acc * (1.0 / _H)


def _attn_dense(q, k, v):
    acc = jnp.zeros((_N, _CH), jnp.float32)
    for h in range(_H):
        sl = slice(h * _CH, (h + 1) * _CH)
        qh, kh, vh = q[:, sl], k[:, sl], v[:, sl]
        s = lax.dot_general(qh, kh, (((1,), (1,)), ((), ())),
                            preferred_element_type=jnp.float32) * _SCALE
        amax = jnp.max(s, axis=1, keepdims=True)
        ex = jnp.exp(s - amax)
        den = jnp.sum(ex, axis=1, keepdims=True)
        o = jnp.dot(ex, vh, preferred_element_type=jnp.float32) / (den + 1e-16)
        acc = acc + o
    return acc * (1.0 / _H)


def _conv_self(x, a, wq, bq, wk, bk, wv, bv, ws, bs):
    q = _proj(x, wq, bq)
    k = _proj(x, wk, bk)
    v = _proj(x, wv, bv)
    o = _attn_masked(q, k, v, a)
    return o + _proj(x, ws, bs)


def _conv_cross(xd, xs, wq, bq, wk, bk, wv, bv, ws, bs):
    q = _proj(xd, wq, bq)
    k = _proj(xs, wk, bk)
    v = _proj(xs, wv, bv)
    o = _attn_dense(q, k, v)
    return o + _proj(xd, ws, bs)


def _l2norm(x):
    n = jnp.sqrt(jnp.sum(x * x, axis=1, keepdims=True))
    return x / jnp.maximum(n, 1e-12)


def _body(x1_ref, x2_ref, a1_ref, a2_ref, *refs):
    cp = refs[:32]
    mp = refs[32:40]
    x1p_ref, x2p_ref, out_ref = refs[40:]

    x1 = x1_ref[...]
    x2 = x2_ref[...]
    a1 = a1_ref[...][:, :_N]
    a2 = a2_ref[...][:, :_N]

    x1 = _conv_self(x1, a1, *cp[0:8])
    x2 = _conv_self(x2, a2, *cp[8:16])
    x1c = _conv_cross(x1, x2, *cp[16:24])
    x2c = _conv_cross(x2, x1, *cp[24:32])
    x1n = _l2norm(x1c)
    x2n = _l2norm(x2c)

    x1p = jnp.mean(x1n, axis=0, keepdims=True)
    x2p = jnp.mean(x2n, axis=0, keepdims=True)
    h = jnp.concatenate([x1p, x2p], axis=1)
    h = jnp.maximum(jnp.dot(h, mp[0][...], preferred_element_type=jnp.float32)
                    + mp[1][...].reshape(1, -1), 0.0)
    h = jnp.maximum(jnp.dot(h, mp[2][...], preferred_element_type=jnp.float32)
                    + mp[3][...].reshape(1, -1), 0.0)
    h = jnp.maximum(jnp.dot(h, mp[4][...], preferred_element_type=jnp.float32)
                    + mp[5][...].reshape(1, -1), 0.0)
    z = jnp.dot(h, mp[6][...], preferred_element_type=jnp.float32) + mp[7][...].reshape(1, -1)
    o = 1.0 / (1.0 + jnp.exp(-z))

    x1p_ref[...] = x1p.reshape(_CH)
    x2p_ref[...] = x2p.reshape(_CH)
    out_ref[...] = o.reshape(1)


def _conv_args(p):
    return [p['Wq'], p['bq'], p['Wk'], p['bk'],
            p['Wv'], p['bv'], p['Ws'], p['bs']]


def kernel(x_1, x_2_pos, edge_index_1, edge_index_2_pos, edge_attr_1,
           edge_attr_2_pos, params):
    lp = params['layers'][0]
    m = params['mlp']
    a1, a2 = _adjacency_sc(edge_index_1.astype(jnp.int32),
                           edge_index_2_pos.astype(jnp.int32))
    args = [x_1, x_2_pos, a1, a2]
    for name in ('text_self', 'graph_self', 'text_cross', 'graph_cross'):
        args.extend(_conv_args(lp[name]))
    args.extend([m['W1'], m['b1'], m['W2'], m['b2'],
                 m['W3'], m['b3'], m['W4'], m['b4']])

    x1p, x2p, out = pl.pallas_call(
        _body,
        out_shape=[
            jax.ShapeDtypeStruct((_CH,), jnp.float32),
            jax.ShapeDtypeStruct((_CH,), jnp.float32),
            jax.ShapeDtypeStruct((1,), jnp.float32),
        ],
        compiler_params=pltpu.CompilerParams(
            vmem_limit_bytes=100 * 1024 * 1024),
    )(*args)
    return x1p, x2p, out
